# baseline (speedup 1.0000x reference)
import numpy as np
import jax
import jax.numpy as jnp
from jax.experimental import pallas as pl
from jax.experimental.pallas import tpu as pltpu

_B = 64
_N = 1000000
_W = 8192
_NB = (_N + _W - 1) // _W

_K0 = np.uint32(0)
_K1 = np.uint32(42)
_K2 = np.uint32(np.uint32(0x1BD11BDA) ^ _K0 ^ _K1)
_KS = (_K0, _K1, _K2)
_ROT = ((13, 15, 26, 6), (17, 29, 16, 24))


def _threefry_bits(x1):
    x0 = x1 * np.uint32(0)
    x1 = x1 + _K1
    for i in range(5):
        for r in _ROT[i % 2]:
            x0 = x0 + x1
            x1 = (x1 << np.uint32(r)) | (x1 >> np.uint32(32 - r))
            x1 = x1 ^ x0
        x0 = x0 + _KS[(i + 1) % 3]
        x1 = x1 + np.uint32(_KS[(i + 2) % 3] + np.uint32(i + 1))
    return x0 ^ x1


def _body(logits_ref, out_ref, bestv_ref, besti_ref):
    j = pl.program_id(0)

    @pl.when(j == 0)
    def _init():
        bestv_ref[...] = jnp.full((_B, 1), -jnp.inf, jnp.float32)
        besti_ref[...] = jnp.zeros((_B, 1), jnp.int32)

    row = jax.lax.broadcasted_iota(jnp.uint32, (_B, _W), 0)
    lane = jax.lax.broadcasted_iota(jnp.uint32, (_B, _W), 1)
    col = jnp.uint32(_W) * j.astype(jnp.uint32) + lane
    flat = row * jnp.uint32(_N) + col

    bits = _threefry_bits(flat)

    fl = jax.lax.bitcast_convert_type(
        (bits >> np.uint32(9)) | np.uint32(0x3F800000), jnp.float32
    ) - np.float32(1.0)
    tiny = np.float32(np.finfo(np.float32).tiny)
    u = jnp.maximum(tiny, fl * (np.float32(1.0) - tiny) + tiny)
    g = -jnp.log(-jnp.log(u))

    s = logits_ref[...] + g
    valid = col < jnp.uint32(_N)
    s = jnp.where(valid, s, -jnp.inf)

    m = jnp.max(s, axis=1, keepdims=True)
    idx = jnp.min(
        jnp.where(s == m, col.astype(jnp.int32), jnp.int32(2**31 - 1)),
        axis=1,
        keepdims=True,
    )
    upd = m > bestv_ref[...]
    besti_ref[...] = jnp.where(upd, idx, besti_ref[...])
    bestv_ref[...] = jnp.where(upd, m, bestv_ref[...])

    @pl.when(j == _NB - 1)
    def _emit():
        out_ref[...] = besti_ref[...]


def kernel(logits):
    out = pl.pallas_call(
        _body,
        grid=(_NB,),
        in_specs=[pl.BlockSpec((_B, _W), lambda j: (0, j))],
        out_specs=pl.BlockSpec((_B, 1), lambda j: (0, 0)),
        out_shape=jax.ShapeDtypeStruct((_B, 1), jnp.int32),
        scratch_shapes=[
            pltpu.VMEM((_B, 1), jnp.float32),
            pltpu.VMEM((_B, 1), jnp.int32),
        ],
    )(logits)
    return out.reshape(_B)


# speedup vs baseline: 1.4493x; 1.4493x over previous
import numpy as np
import jax
import jax.numpy as jnp
from jax.experimental import pallas as pl
from jax.experimental.pallas import tpu as pltpu

_B = 64
_N = 1000000
_W = 8192
_NB = (_N + _W - 1) // _W
_SW = 512
_NSUB = _W // _SW

_K0 = np.uint32(0)
_K1 = np.uint32(42)
_K2 = np.uint32(np.uint32(0x1BD11BDA) ^ _K0 ^ _K1)
_KS = (_K0, _K1, _K2)
_ROT = ((13, 15, 26, 6), (17, 29, 16, 24))


def _threefry_bits(x1):
    x0 = x1 * np.uint32(0)
    x1 = x1 + _K1
    for i in range(5):
        for r in _ROT[i % 2]:
            x0 = x0 + x1
            x1 = (x1 << np.uint32(r)) | (x1 >> np.uint32(32 - r))
            x1 = x1 ^ x0
        x0 = x0 + _KS[(i + 1) % 3]
        x1 = x1 + np.uint32(_KS[(i + 2) % 3] + np.uint32(i + 1))
    return x0 ^ x1


def _body(logits_ref, out_ref, bestv_ref, besti_ref):
    j = pl.program_id(0)

    @pl.when(j == 0)
    def _init():
        bestv_ref[...] = jnp.full((_B, 1), -jnp.inf, jnp.float32)
        besti_ref[...] = jnp.zeros((_B, 1), jnp.int32)

    tiny = np.float32(np.finfo(np.float32).tiny)
    rowbase = jax.lax.broadcasted_iota(jnp.uint32, (_B, _SW), 0) * jnp.uint32(_N)
    lane = jax.lax.broadcasted_iota(jnp.uint32, (_B, _SW), 1)
    jbase = jnp.uint32(_W) * j.astype(jnp.uint32)

    bv = jnp.full((_B, 1), -jnp.inf, jnp.float32)
    bi = jnp.full((_B, 1), jnp.int32(0), jnp.int32)
    for k in range(_NSUB):
        col = lane + (jbase + jnp.uint32(k * _SW))
        bits = _threefry_bits(rowbase + col)
        fl = jax.lax.bitcast_convert_type(
            (bits >> np.uint32(9)) | np.uint32(0x3F800000), jnp.float32
        ) - np.float32(1.0)
        g = -jnp.log(-jnp.log(fl + tiny))

        s = logits_ref[:, k * _SW:(k + 1) * _SW] + g
        s = jnp.where(col < jnp.uint32(_N), s, -jnp.inf)

        m = jnp.max(s, axis=1, keepdims=True)
        idx = jnp.min(
            jnp.where(s == m, col.astype(jnp.int32), jnp.int32(2**31 - 1)),
            axis=1,
            keepdims=True,
        )
        upd = m > bv
        bi = jnp.where(upd, idx, bi)
        bv = jnp.where(upd, m, bv)

    upd = bv > bestv_ref[...]
    besti_ref[...] = jnp.where(upd, bi, besti_ref[...])
    bestv_ref[...] = jnp.where(upd, bv, bestv_ref[...])

    @pl.when(j == _NB - 1)
    def _emit():
        out_ref[...] = besti_ref[...]


def kernel(logits):
    out = pl.pallas_call(
        _body,
        grid=(_NB,),
        in_specs=[pl.BlockSpec((_B, _W), lambda j: (0, j))],
        out_specs=pl.BlockSpec((_B, 1), lambda j: (0, 0)),
        out_shape=jax.ShapeDtypeStruct((_B, 1), jnp.int32),
        scratch_shapes=[
            pltpu.VMEM((_B, 1), jnp.float32),
            pltpu.VMEM((_B, 1), jnp.int32),
        ],
    )(logits)
    return out.reshape(_B)
